# Initial kernel scaffold; baseline (speedup 1.0000x reference)
import jax
import jax.numpy as jnp
from jax.experimental import pallas as pl


def kernel(x_hidden, edge_index, W_l, W_r, b, gamma, beta):
    raise NotImplementedError("write your pallas kernel here")


# speedup vs baseline: 3.9254x; 3.9254x over previous
import functools

import jax
import jax.numpy as jnp
from jax import lax
from jax.experimental import pallas as pl
from jax.experimental.pallas import tpu as pltpu
from jax.experimental.pallas import tpu_sc as plsc

N_NODES_C = 10000
N_EDGES_C = 320000
D_C = 128

NC = 2
NS = 16
NW = NC * NS
CHUNK = 128
EPW = N_EDGES_C // NW
GRP = 8
NGRP = 10
NCHUNKS = NGRP * GRP
EPW_PAD = NCHUNKS * CHUNK
NACC = 10240
RPT = NACC // NS
DUMP_ROW = N_NODES_C


def _sc_body(x_hbm, srcs_hbm, dsts_hbm, part_hbm, cnt_hbm,
             sidx, didx, rows, cnt, acc, sem):
    c = lax.axis_index("c")
    s = lax.axis_index("s")
    wid = s * NC + c
    base = s * RPT

    zeros16 = jnp.zeros((16,), jnp.float32)
    lane = lax.iota(jnp.int32, 16)

    def _fill(r, _):
        for k in range(8):
            rows[r, pl.ds(k * 16, 16)] = zeros16
        return 0
    lax.fori_loop(0, CHUNK, _fill, 0)

    def _fillc(i, _):
        cnt[pl.ds(i * 16, 16)] = zeros16
        return 0
    lax.fori_loop(0, NACC // 16, _fillc, 0)

    for j in range(RPT // CHUNK):
        pltpu.sync_copy(rows, acc.at[pl.ds(base + j * CHUNK, CHUNK)])

    plsc.subcore_barrier()

    def _group(g, _):
        pltpu.sync_copy(srcs_hbm.at[wid, pl.ds(g * GRP, GRP)], sidx)
        pltpu.sync_copy(dsts_hbm.at[wid, pl.ds(g * GRP, GRP)], didx)
        for k in range(GRP):
            pltpu.async_copy(x_hbm.at[sidx.at[k]], rows, sem).wait()
            pltpu.sync_copy(rows, acc.at[didx.at[k]], add=True)

        def _cvec(m, _):
            d16 = didx[m // (CHUNK // 16), pl.ds((m % (CHUNK // 16)) * 16, 16)]
            for lp in range(16):
                d = d16[lp]
                off = (d // 16) * 16
                vec = cnt[pl.ds(off, 16)]
                cnt[pl.ds(off, 16)] = vec + jnp.where(
                    lane == d - off, 1.0, 0.0).astype(jnp.float32)
            return 0
        lax.fori_loop(0, GRP * CHUNK // 16, _cvec, 0)
        return 0
    lax.fori_loop(0, NGRP, _group, 0)

    plsc.subcore_barrier()

    for j in range(RPT // CHUNK):
        pltpu.sync_copy(acc.at[pl.ds(base + j * CHUNK, CHUNK)], rows)
        pltpu.sync_copy(rows, part_hbm.at[c, pl.ds(base + j * CHUNK, CHUNK)])
    pltpu.sync_copy(cnt, cnt_hbm.at[wid])


def _dense_body(part_ref, cnt_ref, x_ref, wl_ref, wr_ref, b_ref, g_ref,
                be_ref, o_ref):
    p = part_ref[0] + part_ref[1]
    cnt = jnp.sum(cnt_ref[...], axis=1, keepdims=True)
    mean_agg = p / jnp.maximum(cnt, 1.0)
    x = x_ref[...]
    h = (jnp.dot(mean_agg, wl_ref[...], precision=lax.Precision.HIGHEST)
         + jnp.dot(x, wr_ref[...], precision=lax.Precision.HIGHEST)
         + b_ref[...])
    h = 0.5 * h * (1.0 + lax.erf(h * (2.0 ** -0.5)))
    y = h + x
    mu = jnp.mean(y, axis=-1, keepdims=True)
    var = jnp.mean((y - mu) ** 2, axis=-1, keepdims=True)
    o_ref[...] = (y - mu) * lax.rsqrt(var + 1e-5) * g_ref[...] + be_ref[...]


def kernel(x_hidden, edge_index, W_l, W_r, b, gamma, beta):
    src = edge_index[0].astype(jnp.int32).reshape(NW, EPW)
    dst = edge_index[1].astype(jnp.int32).reshape(NW, EPW)
    pad = EPW_PAD - EPW
    src = jnp.pad(src, ((0, 0), (0, pad))).reshape(NW, NCHUNKS, CHUNK)
    dst = jnp.pad(dst, ((0, 0), (0, pad)),
                  constant_values=DUMP_ROW).reshape(NW, NCHUNKS, CHUNK)

    mesh = plsc.VectorSubcoreMesh(core_axis_name="c", subcore_axis_name="s")
    part, cnt = pl.kernel(
        _sc_body,
        out_type=(
            jax.ShapeDtypeStruct((NC, NACC, D_C), jnp.float32),
            jax.ShapeDtypeStruct((NW, NACC), jnp.float32),
        ),
        mesh=mesh,
        scratch_types=[
            pltpu.VMEM((GRP, CHUNK), jnp.int32),
            pltpu.VMEM((GRP, CHUNK), jnp.int32),
            pltpu.VMEM((CHUNK, D_C), jnp.float32),
            pltpu.VMEM((NACC,), jnp.float32),
            pltpu.VMEM_SHARED((NACC, D_C), jnp.float32),
            pltpu.SemaphoreType.DMA,
        ],
    )(x_hidden, src, dst)
    cnt = cnt.T

    R = 1000
    grid = (N_NODES_C // R,)
    out = pl.pallas_call(
        _dense_body,
        grid=grid,
        in_specs=[
            pl.BlockSpec((NC, R, D_C), lambda i: (0, i, 0)),
            pl.BlockSpec((R, NW), lambda i: (i, 0)),
            pl.BlockSpec((R, D_C), lambda i: (i, 0)),
            pl.BlockSpec((D_C, D_C), lambda i: (0, 0)),
            pl.BlockSpec((D_C, D_C), lambda i: (0, 0)),
            pl.BlockSpec((1, D_C), lambda i: (0, 0)),
            pl.BlockSpec((1, D_C), lambda i: (0, 0)),
            pl.BlockSpec((1, D_C), lambda i: (0, 0)),
        ],
        out_specs=pl.BlockSpec((R, D_C), lambda i: (i, 0)),
        out_shape=jax.ShapeDtypeStruct((N_NODES_C, D_C), jnp.float32),
    )(part, cnt, x_hidden, W_l, W_r, b.reshape(1, D_C),
      gamma.reshape(1, D_C), beta.reshape(1, D_C))
    return out


# speedup vs baseline: 4.3272x; 1.1024x over previous
import functools

import jax
import jax.numpy as jnp
from jax import lax
from jax.experimental import pallas as pl
from jax.experimental.pallas import tpu as pltpu
from jax.experimental.pallas import tpu_sc as plsc

N_NODES_C = 10000
N_EDGES_C = 320000
D_C = 128

NC = 2
NS = 16
NW = NC * NS
CHUNK = 128
EPW = N_EDGES_C // NW
GRP = 8
NGRP = 10
NCHUNKS = NGRP * GRP
EPW_PAD = NCHUNKS * CHUNK
NACC = 10240
RPT = NACC // NS
DUMP_ROW = N_NODES_C


def _sc_body(x_hbm, srcs_hbm, dsts_hbm, part_hbm, cnt_hbm,
             sidx, didx, rows, rows2, cnt, acc, semA, semB):
    c = lax.axis_index("c")
    s = lax.axis_index("s")
    wid = s * NC + c
    base = s * RPT

    zeros16 = jnp.zeros((16,), jnp.float32)
    lane = lax.iota(jnp.int32, 16)

    def _fill(r, _):
        for k in range(8):
            rows[r, pl.ds(k * 16, 16)] = zeros16
        return 0
    lax.fori_loop(0, CHUNK, _fill, 0)

    def _fillc(i, _):
        cnt[pl.ds(i * 16, 16)] = zeros16
        return 0
    lax.fori_loop(0, NACC // 16, _fillc, 0)

    for j in range(RPT // CHUNK):
        pltpu.sync_copy(rows, acc.at[pl.ds(base + j * CHUNK, CHUNK)])

    plsc.subcore_barrier()

    def _group(g, _):
        pltpu.sync_copy(srcs_hbm.at[wid, pl.ds(g * GRP, GRP)], sidx)
        pltpu.sync_copy(dsts_hbm.at[wid, pl.ds(g * GRP, GRP)], didx)
        bufs = (rows, rows2)
        sems = (semA, semB)
        descs = [pltpu.async_copy(x_hbm.at[sidx.at[0]], bufs[0], sems[0]),
                 None]
        for k in range(GRP):
            b = k % 2
            if k + 1 < GRP:
                descs[1 - b] = pltpu.async_copy(
                    x_hbm.at[sidx.at[k + 1]], bufs[1 - b], sems[1 - b])
            descs[b].wait()
            pltpu.sync_copy(bufs[b], acc.at[didx.at[k]], add=True)

        def _cvec(m, _):
            d16 = didx[m // (CHUNK // 16), pl.ds((m % (CHUNK // 16)) * 16, 16)]
            for lp in range(16):
                d = d16[lp]
                off = (d // 16) * 16
                vec = cnt[pl.ds(off, 16)]
                cnt[pl.ds(off, 16)] = vec + jnp.where(
                    lane == d - off, 1.0, 0.0).astype(jnp.float32)
            return 0
        lax.fori_loop(0, GRP * CHUNK // 16, _cvec, 0)
        return 0
    lax.fori_loop(0, NGRP, _group, 0)

    plsc.subcore_barrier()

    for j in range(RPT // CHUNK):
        pltpu.sync_copy(acc.at[pl.ds(base + j * CHUNK, CHUNK)], rows)
        pltpu.sync_copy(rows, part_hbm.at[c, pl.ds(base + j * CHUNK, CHUNK)])
    pltpu.sync_copy(cnt, cnt_hbm.at[wid])


def _dense_body(part_ref, cnt_ref, x_ref, wl_ref, wr_ref, b_ref, g_ref,
                be_ref, o_ref):
    p = part_ref[0] + part_ref[1]
    cnt = jnp.sum(cnt_ref[...], axis=1, keepdims=True)
    mean_agg = p / jnp.maximum(cnt, 1.0)
    x = x_ref[...]
    h = (jnp.dot(mean_agg, wl_ref[...], precision=lax.Precision.HIGHEST)
         + jnp.dot(x, wr_ref[...], precision=lax.Precision.HIGHEST)
         + b_ref[...])
    h = 0.5 * h * (1.0 + lax.erf(h * (2.0 ** -0.5)))
    y = h + x
    mu = jnp.mean(y, axis=-1, keepdims=True)
    var = jnp.mean((y - mu) ** 2, axis=-1, keepdims=True)
    o_ref[...] = (y - mu) * lax.rsqrt(var + 1e-5) * g_ref[...] + be_ref[...]


def kernel(x_hidden, edge_index, W_l, W_r, b, gamma, beta):
    src = edge_index[0].astype(jnp.int32).reshape(NW, EPW)
    dst = edge_index[1].astype(jnp.int32).reshape(NW, EPW)
    pad = EPW_PAD - EPW
    src = jnp.pad(src, ((0, 0), (0, pad))).reshape(NW, NCHUNKS, CHUNK)
    dst = jnp.pad(dst, ((0, 0), (0, pad)),
                  constant_values=DUMP_ROW).reshape(NW, NCHUNKS, CHUNK)

    mesh = plsc.VectorSubcoreMesh(core_axis_name="c", subcore_axis_name="s")
    part, cnt = pl.kernel(
        _sc_body,
        out_type=(
            jax.ShapeDtypeStruct((NC, NACC, D_C), jnp.float32),
            jax.ShapeDtypeStruct((NW, NACC), jnp.float32),
        ),
        mesh=mesh,
        scratch_types=[
            pltpu.VMEM((GRP, CHUNK), jnp.int32),
            pltpu.VMEM((GRP, CHUNK), jnp.int32),
            pltpu.VMEM((CHUNK, D_C), jnp.float32),
            pltpu.VMEM((CHUNK, D_C), jnp.float32),
            pltpu.VMEM((NACC,), jnp.float32),
            pltpu.VMEM_SHARED((NACC, D_C), jnp.float32),
            pltpu.SemaphoreType.DMA,
            pltpu.SemaphoreType.DMA,
        ],
    )(x_hidden, src, dst)
    cnt = cnt.T

    R = 1000
    grid = (N_NODES_C // R,)
    out = pl.pallas_call(
        _dense_body,
        grid=grid,
        in_specs=[
            pl.BlockSpec((NC, R, D_C), lambda i: (0, i, 0)),
            pl.BlockSpec((R, NW), lambda i: (i, 0)),
            pl.BlockSpec((R, D_C), lambda i: (i, 0)),
            pl.BlockSpec((D_C, D_C), lambda i: (0, 0)),
            pl.BlockSpec((D_C, D_C), lambda i: (0, 0)),
            pl.BlockSpec((1, D_C), lambda i: (0, 0)),
            pl.BlockSpec((1, D_C), lambda i: (0, 0)),
            pl.BlockSpec((1, D_C), lambda i: (0, 0)),
        ],
        out_specs=pl.BlockSpec((R, D_C), lambda i: (i, 0)),
        out_shape=jax.ShapeDtypeStruct((N_NODES_C, D_C), jnp.float32),
    )(part, cnt, x_hidden, W_l, W_r, b.reshape(1, D_C),
      gamma.reshape(1, D_C), beta.reshape(1, D_C))
    return out
